# baseline (speedup 1.0000x reference)
import functools

import jax
import jax.numpy as jnp
import numpy as np
from jax import lax
from jax.experimental import pallas as pl
from jax.experimental.pallas import tpu as pltpu
from jax.experimental.pallas import tpu_sc as plsc

_B = 4096
_N1 = 10
_N2 = 100
_N3 = 9000
_C = _N1 + _N2 + _N3
_EPS = 1e-07
_LEPS = float(np.log(np.float32(_EPS)))
_LHI = float(np.log(np.float32(1.0) - np.float32(_EPS)))

_BR = 256
_CHUNK = 1024
_K = 1 + _N1 + _N2

_NC = 2
_NS = 16
_NW = _NC * _NS
_ROWS_PER_W = _B // _NW
_LANE = 16
_NFULL = (_C - _LANE) // _LANE


def _sc_index_tables():
    s = np.arange(128)
    ia = np.where(s == 0, 111, np.where(s <= 10, s - 1,
                  np.where(s <= 110, 10 + (s - 11), 111))).astype(np.int32)
    ib = np.where((s >= 11) & (s <= 110), (s - 11) // 10, 111).astype(np.int32)
    c = np.arange(_C)
    idxc = np.where(c < _N1, 0,
                    np.where(c < _N1 + _N2, 1 + (c - _N1) // 10,
                             11 + (c - _N1 - _N2) // 90)).astype(np.int32)
    return ia, ib, idxc


_IA, _IB, _IDXC = _sc_index_tables()


def _sc_body(pred_hbm, ia_hbm, ib_hbm, idx_hbm, out_hbm,
             xrow, orow, tmp, hv, iav, ibv, idxv):
    wid = lax.axis_index("s") * _NC + lax.axis_index("c")
    base = wid * _ROWS_PER_W

    pltpu.sync_copy(ia_hbm, iav)
    pltpu.sync_copy(ib_hbm, ibv)
    pltpu.sync_copy(idx_hbm, idxv)

    lanes = lax.iota(jnp.int32, _LANE)

    def row_body(r):
        row = base + r
        pltpu.sync_copy(pred_hbm.at[row], xrow)

        for k in range(7):
            xh = xrow[pl.ds(16 * k, _LANE)]
            ph = 1.0 / (1.0 + jnp.exp(-xh))
            if k == 6:
                ph = jnp.where(lanes == 15, 1.0, ph)
            tmp[pl.ds(16 * k, _LANE)] = ph

        for k in range(8):
            av = plsc.load_gather(tmp, [iav[pl.ds(16 * k, _LANE)]])
            bv = plsc.load_gather(tmp, [ibv[pl.ds(16 * k, _LANE)]])
            hv[pl.ds(16 * k, _LANE)] = av * bv

        def vec_body(v):
            o = _LANE * v
            x = xrow[pl.ds(o, _LANE)]
            pv = plsc.load_gather(hv, [idxv[pl.ds(o, _LANE)]])
            orow[pl.ds(o, _LANE)] = pv / (1.0 + jnp.exp(-x))

        pl.loop(0, _NFULL + 1)(vec_body)
        o = _C - _LANE
        x = xrow[pl.ds(o, _LANE)]
        pv = plsc.load_gather(hv, [idxv[pl.ds(o, _LANE)]])
        orow[pl.ds(o, _LANE)] = pv / (1.0 + jnp.exp(-x))

        pltpu.sync_copy(orow, out_hbm.at[row])

    pl.loop(0, _ROWS_PER_W)(row_body)


def _make_sc_clone():
    @functools.partial(
        pl.kernel,
        out_type=jax.ShapeDtypeStruct((_B, _C), jnp.float32),
        mesh=plsc.VectorSubcoreMesh(core_axis_name="c", subcore_axis_name="s",
                                    num_cores=_NC, num_subcores=_NS),
        scratch_types=[
            pltpu.VMEM((_C,), jnp.float32),
            pltpu.VMEM((_C,), jnp.float32),
            pltpu.VMEM((128,), jnp.float32),
            pltpu.VMEM((128,), jnp.float32),
            pltpu.VMEM((128,), jnp.int32),
            pltpu.VMEM((128,), jnp.int32),
            pltpu.VMEM((_C,), jnp.int32),
        ],
        compiler_params=pltpu.CompilerParams(needs_layout_passes=False),
    )
    def _sc_clone(pred_hbm, ia_hbm, ib_hbm, idx_hbm, out_hbm, *scratch):
        _sc_body(pred_hbm, ia_hbm, ib_hbm, idx_hbm, out_hbm, *scratch)

    return _sc_clone


def _tc_loss_body(pred_ref, tgt_ref, loss_ref, m_ref):
    i = pl.program_id(0)

    @pl.when(i == 0)
    def _init():
        rr = jax.lax.broadcasted_iota(jnp.int32, (_K, _C), 0)
        cc = jax.lax.broadcasted_iota(jnp.int32, (_K, _C), 1)
        lo = jnp.where(rr == 0, 0, jnp.where(rr <= _N1, 10 * rr, 90 * rr - 880))
        width = jnp.where(rr <= _N1, 10, 90)
        m_ref[...] = ((cc >= lo) & (cc < lo + width)).astype(jnp.bfloat16)
        loss_ref[0, 0] = 0.0

    th = tgt_ref[:, :_N1 + _N2]
    one = jnp.ones((_BR, 1), jnp.float32)
    a = jnp.concatenate([one, th], axis=1).astype(jnp.bfloat16)

    part = jnp.zeros((), jnp.float32)
    for c0 in range(0, _C, _CHUNK):
        w = min(_CHUNK, _C - c0)
        x = pred_ref[:, c0:c0 + w]
        t = tgt_ref[:, c0:c0 + w]
        lp = -jnp.log1p(jnp.exp(-x))
        l1 = jnp.clip(lp, _LEPS, _LHI)
        l2 = jnp.clip(lp - x, _LEPS, _LHI)
        mk = jax.lax.dot_general(a, m_ref[:, c0:c0 + w], (((1,), (0,)), ((), ())),
                                 preferred_element_type=jnp.float32)
        part += jnp.sum(jnp.where(t != 0.0, l1, mk * l2))

    loss_ref[0, 0] += part


@jax.jit
def _run(pred, target):
    clone = _make_sc_clone()(pred, jnp.asarray(_IA), jnp.asarray(_IB),
                             jnp.asarray(_IDXC))
    acc = pl.pallas_call(
        _tc_loss_body,
        grid=(_B // _BR,),
        in_specs=[
            pl.BlockSpec((_BR, _C), lambda i: (i, 0)),
            pl.BlockSpec((_BR, _C), lambda i: (i, 0)),
        ],
        out_specs=pl.BlockSpec((1, 1), lambda i: (0, 0), memory_space=pltpu.SMEM),
        out_shape=jax.ShapeDtypeStruct((1, 1), jnp.float32),
        scratch_shapes=[pltpu.VMEM((_K, _C), jnp.bfloat16)],
    )(pred, target)
    loss = -acc[0, 0] / _B
    return loss, clone


def kernel(pred, target, mode=0):
    loss, clone = _run(pred, target)
    return (loss, clone)


# speedup vs baseline: 1.2857x; 1.2857x over previous
import functools

import jax
import jax.numpy as jnp
import numpy as np
from jax import lax
from jax.experimental import pallas as pl
from jax.experimental.pallas import tpu as pltpu
from jax.experimental.pallas import tpu_sc as plsc

_B = 4096
_N1 = 10
_N2 = 100
_N3 = 9000
_C = _N1 + _N2 + _N3
_EPS = 1e-07
_LEPS = float(np.log(np.float32(_EPS)))
_LHI = float(np.log(np.float32(1.0) - np.float32(_EPS)))

_BR = 256
_CHUNK = 1024
_K = 1 + _N1 + _N2

_NC = 2
_NS = 16
_NW = _NC * _NS
_ROWS_PER_W = _B // _NW
_LANE = 16
_NFULL = (_C - _LANE) // _LANE


def _sc_index_tables():
    s = np.arange(128)
    ia = np.where(s == 0, 111, np.where(s <= 10, s - 1,
                  np.where(s <= 110, 10 + (s - 11), 111))).astype(np.int32)
    ib = np.where((s >= 11) & (s <= 110), (s - 11) // 10, 111).astype(np.int32)
    c = np.arange(_C)
    idxc = np.where(c < _N1, 0,
                    np.where(c < _N1 + _N2, 1 + (c - _N1) // 10,
                             11 + (c - _N1 - _N2) // 90)).astype(np.int32)
    return ia, ib, idxc


_IA, _IB, _IDXC = _sc_index_tables()


def _sc_body(pred_hbm, ia_hbm, ib_hbm, idx_hbm, out_hbm,
             x0, x1, o0, o1, tmp, hv, iav, ibv, idxv,
             sin0, sin1, sout0, sout1):
    xrows = [x0, x1]
    orows = [o0, o1]
    sin = [sin0, sin1]
    sout = [sout0, sout1]
    wid = lax.axis_index("s") * _NC + lax.axis_index("c")
    base = wid * _ROWS_PER_W
    last = base + _ROWS_PER_W - 1

    pltpu.sync_copy(ia_hbm, iav)
    pltpu.sync_copy(ib_hbm, ibv)
    pltpu.sync_copy(idx_hbm, idxv)

    lanes = lax.iota(jnp.int32, _LANE)

    def start_in(row, b):
        pltpu.make_async_copy(pred_hbm.at[row], xrows[b], sin[b]).start()

    def compute(xrow, orow):
        for k in range(7):
            xh = xrow[pl.ds(16 * k, _LANE)]
            ph = 1.0 / (1.0 + jnp.exp(-xh))
            if k == 6:
                ph = jnp.where(lanes == 15, 1.0, ph)
            tmp[pl.ds(16 * k, _LANE)] = ph

        for k in range(8):
            av = plsc.load_gather(tmp, [iav[pl.ds(16 * k, _LANE)]])
            bv = plsc.load_gather(tmp, [ibv[pl.ds(16 * k, _LANE)]])
            hv[pl.ds(16 * k, _LANE)] = av * bv

        def vec_body(v):
            o = _LANE * v
            x = xrow[pl.ds(o, _LANE)]
            pv = plsc.load_gather(hv, [idxv[pl.ds(o, _LANE)]])
            orow[pl.ds(o, _LANE)] = pv / (1.0 + jnp.exp(-x))

        pl.loop(0, _NFULL + 1)(vec_body)
        o = _C - _LANE
        x = xrow[pl.ds(o, _LANE)]
        pv = plsc.load_gather(hv, [idxv[pl.ds(o, _LANE)]])
        orow[pl.ds(o, _LANE)] = pv / (1.0 + jnp.exp(-x))

    start_in(base, 0)

    def ring_body(r):
        for b in range(2):
            row = base + r + b
            start_in(jnp.minimum(row + 1, last), 1 - b)
            pltpu.make_async_copy(pred_hbm.at[row], xrows[b], sin[b]).wait()

            @pl.when(r >= 2)
            def _drain_out():
                pltpu.make_async_copy(orows[b], out_hbm.at[row], sout[b]).wait()

            compute(xrows[b], orows[b])
            pltpu.make_async_copy(orows[b], out_hbm.at[row], sout[b]).start()

    pl.loop(0, _ROWS_PER_W, step=2)(ring_body)

    pltpu.make_async_copy(pred_hbm.at[last], xrows[0], sin[0]).wait()
    pltpu.make_async_copy(orows[0], out_hbm.at[last], sout[0]).wait()
    pltpu.make_async_copy(orows[1], out_hbm.at[last], sout[1]).wait()


def _make_sc_clone():
    @functools.partial(
        pl.kernel,
        out_type=jax.ShapeDtypeStruct((_B, _C), jnp.float32),
        mesh=plsc.VectorSubcoreMesh(core_axis_name="c", subcore_axis_name="s",
                                    num_cores=_NC, num_subcores=_NS),
        scratch_types=[
            pltpu.VMEM((_C,), jnp.float32),
            pltpu.VMEM((_C,), jnp.float32),
            pltpu.VMEM((_C,), jnp.float32),
            pltpu.VMEM((_C,), jnp.float32),
            pltpu.VMEM((128,), jnp.float32),
            pltpu.VMEM((128,), jnp.float32),
            pltpu.VMEM((128,), jnp.int32),
            pltpu.VMEM((128,), jnp.int32),
            pltpu.VMEM((_C,), jnp.int32),
            pltpu.SemaphoreType.DMA,
            pltpu.SemaphoreType.DMA,
            pltpu.SemaphoreType.DMA,
            pltpu.SemaphoreType.DMA,
        ],
        compiler_params=pltpu.CompilerParams(needs_layout_passes=False),
    )
    def _sc_clone(pred_hbm, ia_hbm, ib_hbm, idx_hbm, out_hbm, *scratch):
        _sc_body(pred_hbm, ia_hbm, ib_hbm, idx_hbm, out_hbm, *scratch)

    return _sc_clone


def _tc_loss_body(pred_ref, tgt_ref, loss_ref, m_ref):
    i = pl.program_id(0)

    @pl.when(i == 0)
    def _init():
        rr = jax.lax.broadcasted_iota(jnp.int32, (_K, _C), 0)
        cc = jax.lax.broadcasted_iota(jnp.int32, (_K, _C), 1)
        lo = jnp.where(rr == 0, 0, jnp.where(rr <= _N1, 10 * rr, 90 * rr - 880))
        width = jnp.where(rr <= _N1, 10, 90)
        m_ref[...] = ((cc >= lo) & (cc < lo + width)).astype(jnp.bfloat16)
        loss_ref[0, 0] = 0.0

    th = tgt_ref[:, :_N1 + _N2]
    one = jnp.ones((_BR, 1), jnp.float32)
    a = jnp.concatenate([one, th], axis=1).astype(jnp.bfloat16)

    part = jnp.zeros((), jnp.float32)
    for c0 in range(0, _C, _CHUNK):
        w = min(_CHUNK, _C - c0)
        x = pred_ref[:, c0:c0 + w]
        t = tgt_ref[:, c0:c0 + w]
        lp = -jnp.log1p(jnp.exp(-x))
        l1 = jnp.clip(lp, _LEPS, _LHI)
        l2 = jnp.clip(lp - x, _LEPS, _LHI)
        mk = jax.lax.dot_general(a, m_ref[:, c0:c0 + w], (((1,), (0,)), ((), ())),
                                 preferred_element_type=jnp.float32)
        part += jnp.sum(jnp.where(t != 0.0, l1, mk * l2))

    loss_ref[0, 0] += part


@jax.jit
def _run(pred, target):
    clone = _make_sc_clone()(pred, jnp.asarray(_IA), jnp.asarray(_IB),
                             jnp.asarray(_IDXC))
    acc = pl.pallas_call(
        _tc_loss_body,
        grid=(_B // _BR,),
        in_specs=[
            pl.BlockSpec((_BR, _C), lambda i: (i, 0)),
            pl.BlockSpec((_BR, _C), lambda i: (i, 0)),
        ],
        out_specs=pl.BlockSpec((1, 1), lambda i: (0, 0), memory_space=pltpu.SMEM),
        out_shape=jax.ShapeDtypeStruct((1, 1), jnp.float32),
        scratch_shapes=[pltpu.VMEM((_K, _C), jnp.bfloat16)],
    )(pred, target)
    loss = -acc[0, 0] / _B
    return loss, clone


def kernel(pred, target, mode=0):
    loss, clone = _run(pred, target)
    return (loss, clone)


# speedup vs baseline: 1.4392x; 1.1194x over previous
import functools

import jax
import jax.numpy as jnp
import numpy as np
from jax import lax
from jax.experimental import pallas as pl
from jax.experimental.pallas import tpu as pltpu
from jax.experimental.pallas import tpu_sc as plsc

_B = 4096
_N1 = 10
_N2 = 100
_N3 = 9000
_C = _N1 + _N2 + _N3
_EPS = 1e-07
_LEPS = float(np.log(np.float32(_EPS)))
_LHI = float(np.log(np.float32(1.0) - np.float32(_EPS)))

_BR = 128
_CHUNK = 1024
_K = 1 + _N1 + _N2
_TCB = 28
_SCROWS = _B - _TCB * _BR

_NC = 2
_NS = 16
_NW = _NC * _NS
_ROWS_PER_W = _SCROWS // _NW
_LANE = 16
_NFULL = (_C - _LANE) // _LANE


def _sc_index_tables():
    s = np.arange(128)
    ia = np.where(s == 0, 111, np.where(s <= 10, s - 1,
                  np.where(s <= 110, 10 + (s - 11), 111))).astype(np.int32)
    ib = np.where((s >= 11) & (s <= 110), (s - 11) // 10, 111).astype(np.int32)
    c = np.arange(_C)
    idxc = np.where(c < _N1, 0,
                    np.where(c < _N1 + _N2, 1 + (c - _N1) // 10,
                             11 + (c - _N1 - _N2) // 90)).astype(np.int32)
    return ia, ib, idxc


_IA, _IB, _IDXC = _sc_index_tables()


def _sc_body(pred_hbm, ia_hbm, ib_hbm, idx_hbm, out_hbm,
             x0, x1, o0, o1, tmp, hv, iav, ibv, idxv,
             sin0, sin1, sout0, sout1):
    xrows = [x0, x1]
    orows = [o0, o1]
    sin = [sin0, sin1]
    sout = [sout0, sout1]
    wid = lax.axis_index("s") * _NC + lax.axis_index("c")
    base = _TCB * _BR + wid * _ROWS_PER_W
    last = base + _ROWS_PER_W - 1

    pltpu.sync_copy(ia_hbm, iav)
    pltpu.sync_copy(ib_hbm, ibv)
    pltpu.sync_copy(idx_hbm, idxv)

    lanes = lax.iota(jnp.int32, _LANE)

    def start_in(row, b):
        pltpu.make_async_copy(pred_hbm.at[row], xrows[b], sin[b]).start()

    def compute(xrow, orow):
        for k in range(7):
            xh = xrow[pl.ds(16 * k, _LANE)]
            ph = 1.0 / (1.0 + jnp.exp(-xh))
            if k == 6:
                ph = jnp.where(lanes == 15, 1.0, ph)
            tmp[pl.ds(16 * k, _LANE)] = ph

        for k in range(8):
            av = plsc.load_gather(tmp, [iav[pl.ds(16 * k, _LANE)]])
            bv = plsc.load_gather(tmp, [ibv[pl.ds(16 * k, _LANE)]])
            hv[pl.ds(16 * k, _LANE)] = av * bv

        def vec_body(v):
            o = _LANE * v
            x = xrow[pl.ds(o, _LANE)]
            pv = plsc.load_gather(hv, [idxv[pl.ds(o, _LANE)]])
            orow[pl.ds(o, _LANE)] = pv / (1.0 + jnp.exp(-x))

        pl.loop(0, _NFULL + 1)(vec_body)
        o = _C - _LANE
        x = xrow[pl.ds(o, _LANE)]
        pv = plsc.load_gather(hv, [idxv[pl.ds(o, _LANE)]])
        orow[pl.ds(o, _LANE)] = pv / (1.0 + jnp.exp(-x))

    start_in(base, 0)

    def ring_body(r):
        for b in range(2):
            row = base + r + b
            start_in(jnp.minimum(row + 1, last), 1 - b)
            pltpu.make_async_copy(pred_hbm.at[row], xrows[b], sin[b]).wait()

            @pl.when(r >= 2)
            def _drain_out():
                pltpu.make_async_copy(orows[b], out_hbm.at[row], sout[b]).wait()

            compute(xrows[b], orows[b])
            pltpu.make_async_copy(orows[b], out_hbm.at[row], sout[b]).start()

    pl.loop(0, _ROWS_PER_W, step=2)(ring_body)

    pltpu.make_async_copy(pred_hbm.at[last], xrows[0], sin[0]).wait()
    pltpu.make_async_copy(orows[0], out_hbm.at[last], sout[0]).wait()
    pltpu.make_async_copy(orows[1], out_hbm.at[last], sout[1]).wait()


def _make_sc_fill():
    @functools.partial(
        pl.kernel,
        out_type=(),
        mesh=plsc.VectorSubcoreMesh(core_axis_name="c", subcore_axis_name="s",
                                    num_cores=_NC, num_subcores=_NS),
        scratch_types=[
            pltpu.VMEM((_C,), jnp.float32),
            pltpu.VMEM((_C,), jnp.float32),
            pltpu.VMEM((_C,), jnp.float32),
            pltpu.VMEM((_C,), jnp.float32),
            pltpu.VMEM((128,), jnp.float32),
            pltpu.VMEM((128,), jnp.float32),
            pltpu.VMEM((128,), jnp.int32),
            pltpu.VMEM((128,), jnp.int32),
            pltpu.VMEM((_C,), jnp.int32),
            pltpu.SemaphoreType.DMA,
            pltpu.SemaphoreType.DMA,
            pltpu.SemaphoreType.DMA,
            pltpu.SemaphoreType.DMA,
        ],
        compiler_params=pltpu.CompilerParams(needs_layout_passes=False),
    )
    def _sc_fill(pred_hbm, ia_hbm, ib_hbm, idx_hbm, out_hbm, *scratch):
        _sc_body(pred_hbm, ia_hbm, ib_hbm, idx_hbm, out_hbm, *scratch)

    return _sc_fill


def _tc_body(pred_ref, tgt_ref, out_ref, loss_ref, m_ref):
    i = pl.program_id(0)

    @pl.when(i == 0)
    def _init():
        rr = jax.lax.broadcasted_iota(jnp.int32, (_K, _C), 0)
        cc = jax.lax.broadcasted_iota(jnp.int32, (_K, _C), 1)
        lo = jnp.where(rr == 0, 0, jnp.where(rr <= _N1, 10 * rr, 90 * rr - 880))
        width = jnp.where(rr <= _N1, 10, 90)
        m_ref[...] = ((cc >= lo) & (cc < lo + width)).astype(jnp.bfloat16)
        loss_ref[0, 0] = 0.0

    xh = pred_ref[:, :_N1 + _N2]
    th = tgt_ref[:, :_N1 + _N2]
    ph = 1.0 / (1.0 + jnp.exp(-xh))
    p1 = ph[:, :_N1]
    p2 = ph[:, _N1:]

    rr1 = jax.lax.broadcasted_iota(jnp.int32, (_N1, _N2), 0)
    cc1 = jax.lax.broadcasted_iota(jnp.int32, (_N1, _N2), 1)
    m1 = ((cc1 >= 10 * rr1) & (cc1 < 10 * rr1 + 10)).astype(jnp.float32)
    pv2 = jax.lax.dot_general(p1, m1, (((1,), (0,)), ((), ())),
                              preferred_element_type=jnp.float32)
    clone2 = p2 * pv2

    one = jnp.ones((_BR, 1), jnp.float32)
    s_pv = jnp.concatenate([one, p1, clone2], axis=1)
    s_mk = jnp.concatenate([one, th], axis=1)
    a = jnp.concatenate([s_pv, s_mk], axis=0).astype(jnp.bfloat16)

    part = jnp.zeros((), jnp.float32)
    for c0 in range(0, _C, _CHUNK):
        w = min(_CHUNK, _C - c0)
        x = pred_ref[:, c0:c0 + w]
        t = tgt_ref[:, c0:c0 + w]
        p = 1.0 / (1.0 + jnp.exp(-x))
        lp = jnp.log(p)
        l1 = jnp.clip(lp, _LEPS, _LHI)
        l2 = jnp.clip(lp - x, _LEPS, _LHI)
        pvm = jax.lax.dot_general(a, m_ref[:, c0:c0 + w],
                                  (((1,), (0,)), ((), ())),
                                  preferred_element_type=jnp.float32)
        pv = pvm[:_BR]
        mk = pvm[_BR:]

        @pl.when(i < _TCB)
        def _store():
            out_ref[:, c0:c0 + w] = p * pv

        part += jnp.sum(jnp.where(t != 0.0, l1, mk * l2))

    loss_ref[0, 0] += part


@jax.jit
def _run(pred, target):
    clone_tc, acc = pl.pallas_call(
        _tc_body,
        grid=(_B // _BR,),
        in_specs=[
            pl.BlockSpec((_BR, _C), lambda i: (i, 0)),
            pl.BlockSpec((_BR, _C), lambda i: (i, 0)),
        ],
        out_specs=[
            pl.BlockSpec((_BR, _C),
                         lambda i: (jnp.minimum(i, _TCB - 1), 0)),
            pl.BlockSpec((1, 1), lambda i: (0, 0), memory_space=pltpu.SMEM),
        ],
        out_shape=[
            jax.ShapeDtypeStruct((_B, _C), jnp.float32),
            jax.ShapeDtypeStruct((1, 1), jnp.float32),
        ],
        scratch_shapes=[pltpu.VMEM((_K, _C), jnp.bfloat16)],
    )(pred, target)
    clone_ref = jax.new_ref(clone_tc)
    _make_sc_fill()(pred, jnp.asarray(_IA), jnp.asarray(_IB),
                    jnp.asarray(_IDXC), clone_ref)
    loss = -acc[0, 0] / _B
    return loss, clone_ref[...]


def kernel(pred, target, mode=0):
    loss, clone = _run(pred, target)
    return (loss, clone)
